# baseline (speedup 1.0000x reference)
import functools

import jax
import jax.numpy as jnp
from jax import lax
from jax.experimental import pallas as pl
from jax.experimental.pallas import tpu as pltpu
from jax.experimental.pallas import tpu_sc as plsc

D_OUT = 64
SCALE = 8.0
BLK = 128
VSTAGE_BLK = 256


@functools.cache
def _build_stage(V: int):
    VP = -(-V // VSTAGE_BLK) * VSTAGE_BLK

    def body(in_ref, out_ref):
        blk = in_ref[...]
        y = jnp.transpose(blk) * SCALE
        z = jnp.zeros((VSTAGE_BLK, D_OUT), jnp.float32)
        out_ref[...] = jnp.concatenate([y, z], axis=1)

    return pl.pallas_call(
        body,
        grid=(VP // VSTAGE_BLK,),
        in_specs=[pl.BlockSpec((D_OUT, VSTAGE_BLK), lambda i: (0, i))],
        out_specs=pl.BlockSpec((VSTAGE_BLK, 128), lambda i: (i, 0)),
        out_shape=jax.ShapeDtypeStruct((VP, 128), jnp.float32),
    )


@functools.cache
def _build_gather(B: int, VP: int):
    info = plsc.get_sparse_core_info()
    NC, NS = info.num_cores, info.num_subcores
    NW = NC * NS
    b_per_w = B // NW
    n_blocks = b_per_w // BLK
    mesh = plsc.VectorSubcoreMesh(core_axis_name="c", subcore_axis_name="s")

    def body(x_hbm, tab_hbm, out_hbm, idx_v, r0, r1, r2, r3,
             g0, g1, g2, g3, o0, o1, o2, o3):
        wid = lax.axis_index("s") * NC + lax.axis_index("c")
        base = wid * b_per_w
        pltpu.sync_copy(x_hbm.at[pl.ds(base, b_per_w)], idx_v)
        rows = [r0, r1, r2, r3]
        gsems = [g0, g1, g2, g3]
        osems = [o0, o1, o2, o3]

        def fire_gather(c, s):
            pltpu.async_copy(
                tab_hbm.at[idx_v.at[pl.ds(c * BLK, BLK)]], rows[s], gsems[s]
            )

        def drain_gather(c, s):
            pltpu.make_async_copy(
                tab_hbm.at[idx_v.at[pl.ds(c * BLK, BLK)]], rows[s], gsems[s]
            ).wait()

        def out_copy(c, s, fire):
            src = rows[s].at[pl.ds(0, BLK), pl.ds(0, D_OUT)]
            dst = out_hbm.at[pl.ds(base + c * BLK, BLK)]
            if fire:
                pltpu.async_copy(src, dst, osems[s])
            else:
                pltpu.make_async_copy(src, dst, osems[s]).wait()

        fire_gather(0, 0)
        fire_gather(1, 1)

        def group(g, carry):
            c0 = 4 * g
            for k in range(4):
                c = c0 + k

                @pl.when(c >= 2)
                def _():
                    out_copy(c - 2, (k + 2) % 4, False)

                @pl.when(c + 2 < n_blocks)
                def _():
                    fire_gather(c + 2, (k + 2) % 4)

                drain_gather(c, k)
                out_copy(c, k, True)
            return carry

        lax.fori_loop(0, n_blocks // 4, group, 0)
        out_copy(n_blocks - 2, (n_blocks - 2) % 4, False)
        out_copy(n_blocks - 1, (n_blocks - 1) % 4, False)

    return pl.kernel(
        body,
        out_type=jax.ShapeDtypeStruct((B, D_OUT), jnp.float32),
        mesh=mesh,
        compiler_params=pltpu.CompilerParams(
            use_tc_tiling_on_sc=False,
            needs_layout_passes=False,
            disable_bounds_checks=True,
        ),
        scratch_types=[
            pltpu.VMEM((b_per_w,), jnp.int32),
            pltpu.VMEM((BLK, 128), jnp.float32),
            pltpu.VMEM((BLK, 128), jnp.float32),
            pltpu.VMEM((BLK, 128), jnp.float32),
            pltpu.VMEM((BLK, 128), jnp.float32),
            pltpu.SemaphoreType.DMA,
            pltpu.SemaphoreType.DMA,
            pltpu.SemaphoreType.DMA,
            pltpu.SemaphoreType.DMA,
            pltpu.SemaphoreType.DMA,
            pltpu.SemaphoreType.DMA,
            pltpu.SemaphoreType.DMA,
            pltpu.SemaphoreType.DMA,
        ],
    )


@functools.cache
def _build_trans(S0: int, S1: int):
    def body(in_ref, out_ref):
        blk = in_ref[...]
        b4 = blk.reshape(D_OUT, 2, 8, 8)
        y = jnp.transpose(b4, (2, 3, 0, 1))
        out_ref[0, :, 0, :, :] = y.reshape(8, 8, BLK)

    return pl.pallas_call(
        body,
        grid=(S1, S0 // BLK),
        in_specs=[
            pl.BlockSpec((D_OUT, 128), lambda s1, s0b: (s1 * (S0 // BLK) + s0b, 0))
        ],
        out_specs=pl.BlockSpec(
            (1, 8, 1, 8, BLK), lambda s1, s0b: (s1, 0, s0b, 0, 0)
        ),
        out_shape=jax.ShapeDtypeStruct(
            (S1, D_OUT // 8, S0 // BLK, 8, BLK), jnp.float32
        ),
    )


def kernel(x, table):
    s0, s1 = x.shape
    v, d = table.shape
    xt = x.T.reshape(s0 * s1).astype(jnp.int32)
    staged = _build_stage(v)(table.T)
    flat = _build_gather(s0 * s1, staged.shape[0])(xt, staged)
    out5d = _build_trans(s0, s1)(flat.reshape(s0 * s1 // 2, 128))
    return out5d.transpose(2, 4, 0, 1, 3).reshape(s0, s1, d)


# speedup vs baseline: 3.2705x; 3.2705x over previous
import functools

import jax
import jax.numpy as jnp
from jax import lax
from jax.experimental import pallas as pl
from jax.experimental.pallas import tpu as pltpu
from jax.experimental.pallas import tpu_sc as plsc

D_OUT = 64
SCALE = 8.0
BLK = 128
VSTAGE_BLK = 256


@functools.cache
def _build_stage(V: int):
    VP = -(-V // VSTAGE_BLK) * VSTAGE_BLK

    def body(in_ref, out_ref):
        eye = jnp.eye(D_OUT, dtype=jnp.float32)
        blk = in_ref[...]
        y = lax.dot_general(
            blk, eye * SCALE, (((0,), (0,)), ((), ())),
            preferred_element_type=jnp.float32,
        )
        z = jnp.zeros((VSTAGE_BLK, D_OUT), jnp.float32)
        out_ref[...] = jnp.concatenate([y, z], axis=1)

    return pl.pallas_call(
        body,
        grid=(VP // VSTAGE_BLK,),
        in_specs=[pl.BlockSpec((D_OUT, VSTAGE_BLK), lambda i: (0, i))],
        out_specs=pl.BlockSpec((VSTAGE_BLK, 128), lambda i: (i, 0)),
        out_shape=jax.ShapeDtypeStruct((VP, 128), jnp.float32),
    )


@functools.cache
def _build_gather(B: int, VP: int):
    info = plsc.get_sparse_core_info()
    NC, NS = info.num_cores, info.num_subcores
    NW = NC * NS
    b_per_w = B // NW
    n_blocks = b_per_w // BLK
    mesh = plsc.VectorSubcoreMesh(core_axis_name="c", subcore_axis_name="s")

    def body(x_hbm, tab_hbm, out_hbm, idx_v, r0, r1, r2, r3,
             g0, g1, g2, g3, o0, o1, o2, o3):
        wid = lax.axis_index("s") * NC + lax.axis_index("c")
        base = wid * b_per_w
        pltpu.sync_copy(x_hbm.at[pl.ds(base, b_per_w)], idx_v)
        rows = [r0, r1, r2, r3]
        gsems = [g0, g1, g2, g3]
        osems = [o0, o1, o2, o3]

        def fire_gather(c, s):
            pltpu.async_copy(
                tab_hbm.at[idx_v.at[pl.ds(c * BLK, BLK)]], rows[s], gsems[s]
            )

        def drain_gather(c, s):
            pltpu.make_async_copy(
                tab_hbm.at[idx_v.at[pl.ds(c * BLK, BLK)]], rows[s], gsems[s]
            ).wait()

        def out_copy(c, s, fire):
            src = rows[s].at[pl.ds(0, BLK), pl.ds(0, D_OUT)]
            dst = out_hbm.at[pl.ds(base + c * BLK, BLK)]
            if fire:
                pltpu.async_copy(src, dst, osems[s])
            else:
                pltpu.make_async_copy(src, dst, osems[s]).wait()

        fire_gather(0, 0)
        fire_gather(1, 1)

        def group(g, carry):
            c0 = 4 * g
            for k in range(4):
                c = c0 + k

                @pl.when(c >= 2)
                def _():
                    out_copy(c - 2, (k + 2) % 4, False)

                @pl.when(c + 2 < n_blocks)
                def _():
                    fire_gather(c + 2, (k + 2) % 4)

                drain_gather(c, k)
                out_copy(c, k, True)
            return carry

        lax.fori_loop(0, n_blocks // 4, group, 0)
        out_copy(n_blocks - 2, (n_blocks - 2) % 4, False)
        out_copy(n_blocks - 1, (n_blocks - 1) % 4, False)

    return pl.kernel(
        body,
        out_type=jax.ShapeDtypeStruct((B, D_OUT), jnp.float32),
        mesh=mesh,
        compiler_params=pltpu.CompilerParams(
            use_tc_tiling_on_sc=False,
            needs_layout_passes=False,
            disable_bounds_checks=True,
        ),
        scratch_types=[
            pltpu.VMEM((b_per_w,), jnp.int32),
            pltpu.VMEM((BLK, 128), jnp.float32),
            pltpu.VMEM((BLK, 128), jnp.float32),
            pltpu.VMEM((BLK, 128), jnp.float32),
            pltpu.VMEM((BLK, 128), jnp.float32),
            pltpu.SemaphoreType.DMA,
            pltpu.SemaphoreType.DMA,
            pltpu.SemaphoreType.DMA,
            pltpu.SemaphoreType.DMA,
            pltpu.SemaphoreType.DMA,
            pltpu.SemaphoreType.DMA,
            pltpu.SemaphoreType.DMA,
            pltpu.SemaphoreType.DMA,
        ],
    )


@functools.cache
def _build_trans(S0: int, S1: int):
    NB = 4

    def body(in_ref, out_ref):
        eye = jnp.eye(D_OUT, dtype=jnp.float32)
        for b in range(NB):
            blk = in_ref[pl.ds(b * D_OUT, D_OUT), :]
            halves = [
                lax.dot_general(
                    blk[:, h * 64:(h + 1) * 64], eye,
                    (((0,), (0,)), ((), ())),
                    preferred_element_type=jnp.float32,
                )
                for h in range(2)
            ]
            y = jnp.concatenate(halves, axis=1)
            out_ref[0, :, b, :, :] = y.reshape(8, 8, BLK)

    return pl.pallas_call(
        body,
        grid=(S1, S0 // (BLK * NB)),
        in_specs=[
            pl.BlockSpec(
                (D_OUT * NB, 128),
                lambda s1, g: (s1 * (S0 // (BLK * NB)) + g, 0),
            )
        ],
        out_specs=pl.BlockSpec(
            (1, 8, NB, 8, BLK), lambda s1, g: (s1, 0, g, 0, 0)
        ),
        out_shape=jax.ShapeDtypeStruct(
            (S1, D_OUT // 8, S0 // BLK, 8, BLK), jnp.float32
        ),
    )


def kernel(x, table):
    s0, s1 = x.shape
    v, d = table.shape
    xt = x.T.reshape(s0 * s1).astype(jnp.int32)
    xt = xt.reshape(-1, 2, 64).transpose(0, 2, 1).reshape(s0 * s1)
    staged = _build_stage(v)(table.T)
    flat = _build_gather(s0 * s1, staged.shape[0])(xt, staged)
    out5d = _build_trans(s0, s1)(flat.reshape(s0 * s1 // 2, 128))
    return out5d.transpose(2, 4, 0, 1, 3).reshape(s0, s1, d)


# speedup vs baseline: 7.7990x; 2.3847x over previous
import functools

import jax
import jax.numpy as jnp
from jax import lax
from jax.experimental import pallas as pl
from jax.experimental.pallas import tpu as pltpu
from jax.experimental.pallas import tpu_sc as plsc

D_OUT = 64
SCALE = 8.0
BLK = 128
VSTAGE_BLK = 2048


@functools.cache
def _build_stage(V: int):
    VP = -(-V // VSTAGE_BLK) * VSTAGE_BLK

    def body(in_ref, out_ref):
        eye = jnp.eye(D_OUT, dtype=jnp.float32)
        blk = in_ref[...]
        y = lax.dot_general(
            blk, eye * SCALE, (((0,), (0,)), ((), ())),
            preferred_element_type=jnp.float32,
        )
        z = jnp.zeros((VSTAGE_BLK, D_OUT), jnp.float32)
        out_ref[...] = jnp.concatenate([y, z], axis=1)

    return pl.pallas_call(
        body,
        grid=(VP // VSTAGE_BLK,),
        in_specs=[pl.BlockSpec((D_OUT, VSTAGE_BLK), lambda i: (0, i))],
        out_specs=pl.BlockSpec((VSTAGE_BLK, 128), lambda i: (i, 0)),
        out_shape=jax.ShapeDtypeStruct((VP, 128), jnp.float32),
    )


@functools.cache
def _build_gather(B: int, VP: int):
    info = plsc.get_sparse_core_info()
    NC, NS = info.num_cores, info.num_subcores
    NW = NC * NS
    b_per_w = B // NW
    n_blocks = b_per_w // BLK
    mesh = plsc.VectorSubcoreMesh(core_axis_name="c", subcore_axis_name="s")

    def body(x_hbm, tab_hbm, out_hbm, idx_v, r0, r1, r2, r3,
             g0, g1, g2, g3, o0, o1, o2, o3):
        wid = lax.axis_index("s") * NC + lax.axis_index("c")
        base = wid * b_per_w
        pltpu.sync_copy(x_hbm.at[pl.ds(base, b_per_w)], idx_v)
        rows = [r0, r1, r2, r3]
        gsems = [g0, g1, g2, g3]
        osems = [o0, o1, o2, o3]

        def fire_gather(c, s):
            pltpu.async_copy(
                tab_hbm.at[idx_v.at[pl.ds(c * BLK, BLK)]], rows[s], gsems[s]
            )

        def drain_gather(c, s):
            pltpu.make_async_copy(
                tab_hbm.at[idx_v.at[pl.ds(c * BLK, BLK)]], rows[s], gsems[s]
            ).wait()

        def out_copy(c, s, fire):
            src = rows[s].at[pl.ds(0, BLK), pl.ds(0, D_OUT)]
            dst = out_hbm.at[pl.ds(base + c * BLK, BLK)]
            if fire:
                pltpu.async_copy(src, dst, osems[s])
            else:
                pltpu.make_async_copy(src, dst, osems[s]).wait()

        fire_gather(0, 0)
        fire_gather(1, 1)

        def group(g, carry):
            c0 = 4 * g
            for k in range(4):
                c = c0 + k

                @pl.when(c >= 2)
                def _():
                    out_copy(c - 2, (k + 2) % 4, False)

                @pl.when(c + 2 < n_blocks)
                def _():
                    fire_gather(c + 2, (k + 2) % 4)

                drain_gather(c, k)
                out_copy(c, k, True)
            return carry

        lax.fori_loop(0, n_blocks // 4, group, 0)
        out_copy(n_blocks - 2, (n_blocks - 2) % 4, False)
        out_copy(n_blocks - 1, (n_blocks - 1) % 4, False)

    return pl.kernel(
        body,
        out_type=jax.ShapeDtypeStruct((B, D_OUT), jnp.float32),
        mesh=mesh,
        compiler_params=pltpu.CompilerParams(
            use_tc_tiling_on_sc=False,
            needs_layout_passes=False,
            disable_bounds_checks=True,
        ),
        scratch_types=[
            pltpu.VMEM((b_per_w,), jnp.int32),
            pltpu.VMEM((BLK, 128), jnp.float32),
            pltpu.VMEM((BLK, 128), jnp.float32),
            pltpu.VMEM((BLK, 128), jnp.float32),
            pltpu.VMEM((BLK, 128), jnp.float32),
            pltpu.SemaphoreType.DMA,
            pltpu.SemaphoreType.DMA,
            pltpu.SemaphoreType.DMA,
            pltpu.SemaphoreType.DMA,
            pltpu.SemaphoreType.DMA,
            pltpu.SemaphoreType.DMA,
            pltpu.SemaphoreType.DMA,
            pltpu.SemaphoreType.DMA,
        ],
    )


@functools.cache
def _build_trans(S0: int, S1: int):
    NB = 16

    def body(in_ref, out_ref):
        eye = jnp.eye(D_OUT, dtype=jnp.float32)
        for b in range(NB):
            blk = in_ref[pl.ds(b * D_OUT, D_OUT), :]
            halves = [
                lax.dot_general(
                    blk[:, h * 64:(h + 1) * 64], eye,
                    (((0,), (0,)), ((), ())),
                    preferred_element_type=jnp.float32,
                )
                for h in range(2)
            ]
            y = jnp.concatenate(halves, axis=1)
            out_ref[0, :, b, :, :] = y.reshape(8, 8, BLK)

    return pl.pallas_call(
        body,
        grid=(S1, S0 // (BLK * NB)),
        in_specs=[
            pl.BlockSpec(
                (D_OUT * NB, 128),
                lambda s1, g: (s1 * (S0 // (BLK * NB)) + g, 0),
            )
        ],
        out_specs=pl.BlockSpec(
            (1, 8, NB, 8, BLK), lambda s1, g: (s1, 0, g, 0, 0)
        ),
        out_shape=jax.ShapeDtypeStruct(
            (S1, D_OUT // 8, S0 // BLK, 8, BLK), jnp.float32
        ),
    )


def kernel(x, table):
    s0, s1 = x.shape
    v, d = table.shape
    xt = x.T.reshape(s0 * s1).astype(jnp.int32)
    xt = xt.reshape(-1, 2, 64).transpose(0, 2, 1).reshape(s0 * s1)
    staged = _build_stage(v)(table.T)
    flat = _build_gather(s0 * s1, staged.shape[0])(xt, staged)
    out5d = _build_trans(s0, s1)(flat.reshape(s0 * s1 // 2, 128))
    return out5d.transpose(2, 4, 0, 1, 3).reshape(s0, s1, d)
